# baseline (speedup 1.0000x reference)
import functools

import jax
import jax.numpy as jnp
from jax import lax
from jax.experimental import pallas as pl
from jax.experimental.pallas import tpu as pltpu
from jax.experimental.pallas import tpu_sc as plsc

N = 10000
D = 128
H = 16
C = 6
E = 320000

NPAD = 10240
DUMMY = NPAD - 1
NW = 32
B = 128
NB = 84
EPW = NB * B
ETP = NW * EPW
RPT = NPAD // 16

def _edge_pass_body(src_hbm, dst_hbm, xnh_hbm, xnb_hbm, out_hbm,
                    sidx, didx, srows0, drows0, srows1, drows1,
                    srows2, drows2, srows3, drows3,
                    contrib0, contrib1, contrib2, contrib3, acc,
                    gs0, gd0, gs1, gd1, gs2, gd2, gs3, gd3,
                    sc0, sc1, sc2, sc3):
    c = lax.axis_index("c")
    s = lax.axis_index("s")
    wid = c * 16 + s

    def _zero(j, carry):
        contrib0[j, pl.ds(0, 16)] = jnp.zeros((16,), jnp.float32)
        contrib0[j, pl.ds(16, 16)] = jnp.zeros((16,), jnp.float32)
        return carry

    lax.fori_loop(0, B, _zero, 0)
    base = s * RPT
    for k in range(RPT // B):
        pltpu.sync_copy(contrib0, acc.at[pl.ds(base + k * B, B)])
    plsc.subcore_barrier()

    pltpu.sync_copy(src_hbm.at[wid], sidx)
    pltpu.sync_copy(dst_hbm.at[wid], didx)

    bufs = ((srows0, drows0, gs0, gd0, contrib0, sc0),
            (srows1, drows1, gs1, gd1, contrib1, sc1),
            (srows2, drows2, gs2, gd2, contrib2, sc2),
            (srows3, drows3, gs3, gd3, contrib3, sc3))

    def _fire(b, p):
        srows, drows, ss, sd, _, _ = bufs[p]
        pltpu.async_copy(xnh_hbm.at[sidx.at[b]], srows, ss)
        pltpu.async_copy(xnb_hbm.at[didx.at[b]], drows, sd)

    def _wait_gather(p):
        srows, drows, ss, sd, _, _ = bufs[p]
        pltpu.make_async_copy(xnh_hbm.at[sidx.at[0]], srows, ss).wait()
        pltpu.make_async_copy(xnb_hbm.at[didx.at[0]], drows, sd).wait()

    def _consume(b, p):
        srows, drows, _, _, contrib, sc = bufs[p]

        @functools.partial(plsc.parallel_loop, 0, B, unroll=8)
        def _edge(j):
            xs = srows[j, pl.ds(0, 16)]
            hs = srows[j, pl.ds(16, 16)]
            xd = drows[j]
            a = jnp.sum(xs * xd)
            wv = jnp.exp(jnp.full((16,), a, jnp.float32))
            contrib[j, pl.ds(0, 16)] = wv * hs
            contrib[j, pl.ds(16, 16)] = wv

        pltpu.async_copy(contrib, acc.at[didx.at[b]], sc, add=True).wait()

    for p in range(3):
        _fire(p, p)

    def _outer(i, carry):
        for p in range(4):
            b = 4 * i + p

            @pl.when(b + 3 < NB)
            def _():
                _fire(b + 3, (p + 3) % 4)

            _wait_gather(p)
            _consume(b, p)
        return carry

    lax.fori_loop(0, NB // 4, _outer, 0)
    plsc.subcore_barrier()

    pltpu.sync_copy(acc.at[pl.ds(base, RPT)], out_hbm.at[c, pl.ds(base, RPT)])


@functools.cache
def _edge_pass():
    mesh = plsc.VectorSubcoreMesh(core_axis_name="c", subcore_axis_name="s")
    return pl.kernel(
        _edge_pass_body,
        out_type=jax.ShapeDtypeStruct((2, NPAD, 32), jnp.float32),
        mesh=mesh,
        scratch_types=[
            pltpu.VMEM((NB, B), jnp.int32),
            pltpu.VMEM((NB, B), jnp.int32),
            pltpu.VMEM((B, 32), jnp.float32),
            pltpu.VMEM((B, 16), jnp.float32),
            pltpu.VMEM((B, 32), jnp.float32),
            pltpu.VMEM((B, 16), jnp.float32),
            pltpu.VMEM((B, 32), jnp.float32),
            pltpu.VMEM((B, 16), jnp.float32),
            pltpu.VMEM((B, 32), jnp.float32),
            pltpu.VMEM((B, 16), jnp.float32),
            pltpu.VMEM((B, 32), jnp.float32),
            pltpu.VMEM((B, 32), jnp.float32),
            pltpu.VMEM((B, 32), jnp.float32),
            pltpu.VMEM((B, 32), jnp.float32),
            pltpu.VMEM_SHARED((NPAD, 32), jnp.float32),
        ] + [pltpu.SemaphoreType.DMA] * 12,
        compiler_params=pltpu.CompilerParams(
            needs_layout_passes=False, use_tc_tiling_on_sc=False),
    )


def _prep_body(x_ref, w1_ref, b1_ref, xnh_ref, xnb_ref):
    x = x_ref[...]
    h = jnp.maximum(x @ w1_ref[...] + b1_ref[...], 0.0)
    nrm = jnp.sqrt(jnp.sum(h * h, axis=1, keepdims=True))
    xn = h / jnp.maximum(nrm, 1e-12)
    pad2 = jnp.zeros((NPAD - N, 32), jnp.float32)
    pad1 = jnp.zeros((NPAD - N, 16), jnp.float32)
    xnh_ref[...] = jnp.concatenate([jnp.concatenate([xn, h], axis=1), pad2], axis=0)
    xnb_ref[...] = jnp.concatenate([xn, pad1], axis=0)


def _comb_body(nd_ref, beta_ref, x1_ref, xnh_ref, xnb_ref):
    nd = nd_ref[...]
    num = nd[0, :N, 0:16] + nd[1, :N, 0:16]
    den = nd[0, :N, 16:17] + nd[1, :N, 16:17]
    x1 = num / den
    x1_ref[...] = x1
    nrm = jnp.sqrt(jnp.sum(x1 * x1, axis=1, keepdims=True))
    xn = x1 / jnp.maximum(nrm, 1e-12)
    pad2 = jnp.zeros((NPAD - N, 32), jnp.float32)
    pad1 = jnp.zeros((NPAD - N, 16), jnp.float32)
    xnh_ref[...] = jnp.concatenate([jnp.concatenate([xn, x1], axis=1), pad2], axis=0)
    xnb_ref[...] = jnp.concatenate([xn * beta_ref[0, 0], pad1], axis=0)


def _final_body(nd_ref, w2_ref, b2_ref, out_ref):
    nd = nd_ref[...]
    num = nd[0, :N, 0:16] + nd[1, :N, 0:16]
    den = nd[0, :N, 16:17] + nd[1, :N, 16:17]
    h2 = num / den
    logits = h2 @ w2_ref[...] + b2_ref[...]
    m = jnp.max(logits, axis=1, keepdims=True)
    lse = jnp.log(jnp.sum(jnp.exp(logits - m), axis=1, keepdims=True)) + m
    out_ref[...] = logits - lse


_prep = pl.pallas_call(
    _prep_body,
    out_shape=[
        jax.ShapeDtypeStruct((NPAD, 32), jnp.float32),
        jax.ShapeDtypeStruct((NPAD, 16), jnp.float32),
    ],
)

_comb = pl.pallas_call(
    _comb_body,
    out_shape=[
        jax.ShapeDtypeStruct((N, 16), jnp.float32),
        jax.ShapeDtypeStruct((NPAD, 32), jnp.float32),
        jax.ShapeDtypeStruct((NPAD, 16), jnp.float32),
    ],
)

_final = pl.pallas_call(
    _final_body,
    out_shape=jax.ShapeDtypeStruct((N, C), jnp.float32),
)


def kernel(x, edge_index, W1, b1, beta2, W2, b2):
    loop = jnp.arange(N, dtype=jnp.int32)
    padi = jnp.full((ETP - E - N,), DUMMY, dtype=jnp.int32)
    src = jnp.concatenate([edge_index[0], loop, padi]).reshape(NW, NB, B)
    dst = jnp.concatenate([edge_index[1], loop, padi]).reshape(NW, NB, B)

    xnh1, xnb1 = _prep(x, W1, b1.reshape(1, H))
    ep = _edge_pass()
    nd1 = ep(src, dst, xnh1, xnb1)
    x1, xnh2, xnb2 = _comb(nd1, beta2.reshape(1, 1))
    nd2 = ep(src, dst, xnh2, xnb2)
    logp = _final(nd2, W2, b2.reshape(1, C))
    return (logp, x1)


# speedup vs baseline: 2.3562x; 2.3562x over previous
import functools

import jax
import jax.numpy as jnp
from jax import lax
from jax.experimental import pallas as pl
from jax.experimental.pallas import tpu as pltpu
from jax.experimental.pallas import tpu_sc as plsc

N = 10000
D = 128
H = 16
C = 6
E = 320000

NPAD = 10240
DUMMY = NPAD - 1
NW = 32
B = 128
NB = 84
EPW = NB * B
ETP = NW * EPW
RPT = NPAD // 16

def _edge_pass_body(src_hbm, dst_hbm, xnh_hbm, xnb_hbm, out_hbm,
                    sidx, didx, srows0, drows0, srows1, drows1,
                    srows2, drows2, srows3, drows3,
                    contrib0, contrib1, contrib2, contrib3, acc,
                    gs0, gd0, gs1, gd1, gs2, gd2, gs3, gd3,
                    sc0, sc1, sc2, sc3):
    c = lax.axis_index("c")
    s = lax.axis_index("s")
    wid = c * 16 + s

    def _zero(j, carry):
        contrib0[j, pl.ds(0, 16)] = jnp.zeros((16,), jnp.float32)
        contrib0[j, pl.ds(16, 16)] = jnp.zeros((16,), jnp.float32)
        return carry

    lax.fori_loop(0, B, _zero, 0)
    base = s * RPT
    for k in range(RPT // B):
        pltpu.sync_copy(contrib0, acc.at[pl.ds(base + k * B, B)])
    plsc.subcore_barrier()

    pltpu.sync_copy(src_hbm.at[wid], sidx)
    pltpu.sync_copy(dst_hbm.at[wid], didx)

    bufs = ((srows0, drows0, gs0, gd0, contrib0, sc0),
            (srows1, drows1, gs1, gd1, contrib1, sc1),
            (srows2, drows2, gs2, gd2, contrib2, sc2),
            (srows3, drows3, gs3, gd3, contrib3, sc3))

    def _fire(b, p):
        srows, drows, ss, sd, _, _ = bufs[p]
        pltpu.async_copy(xnh_hbm.at[sidx.at[b]], srows, ss)
        pltpu.async_copy(xnb_hbm.at[didx.at[b]], drows, sd)

    def _wait_gather(p):
        srows, drows, ss, sd, _, _ = bufs[p]
        pltpu.make_async_copy(xnh_hbm.at[sidx.at[0]], srows, ss).wait()
        pltpu.make_async_copy(xnb_hbm.at[didx.at[0]], drows, sd).wait()

    def _consume(b, p):
        srows, drows, _, _, contrib, sc = bufs[p]

        @functools.partial(plsc.parallel_loop, 0, B, unroll=8)
        def _edge(j):
            xs = srows[j, pl.ds(0, 16)]
            hs = srows[j, pl.ds(16, 16)]
            xd = drows[j]
            a = jnp.sum(xs * xd)
            wv = jnp.exp(jnp.full((16,), a, jnp.float32))
            contrib[j, pl.ds(0, 16)] = wv * hs
            contrib[j, pl.ds(16, 16)] = wv

        pltpu.async_copy(contrib, acc.at[didx.at[b]], sc, add=True).wait()

    for p in range(3):
        _fire(p, p)

    def _outer(i, carry):
        for p in range(4):
            b = 4 * i + p

            @pl.when(b + 3 < NB)
            def _():
                _fire(b + 3, (p + 3) % 4)

            _wait_gather(p)
            _consume(b, p)
        return carry

    lax.fori_loop(0, NB // 4, _outer, 0)
    plsc.subcore_barrier()

    pltpu.sync_copy(acc.at[pl.ds(base, RPT)], out_hbm.at[c, pl.ds(base, RPT)])


@functools.cache
def _edge_pass():
    mesh = plsc.VectorSubcoreMesh(core_axis_name="c", subcore_axis_name="s")
    return pl.kernel(
        _edge_pass_body,
        out_type=jax.ShapeDtypeStruct((2, NPAD, 32), jnp.float32),
        mesh=mesh,
        scratch_types=[
            pltpu.VMEM((NB, B), jnp.int32),
            pltpu.VMEM((NB, B), jnp.int32),
            pltpu.VMEM((B, 32), jnp.float32),
            pltpu.VMEM((B, 16), jnp.float32),
            pltpu.VMEM((B, 32), jnp.float32),
            pltpu.VMEM((B, 16), jnp.float32),
            pltpu.VMEM((B, 32), jnp.float32),
            pltpu.VMEM((B, 16), jnp.float32),
            pltpu.VMEM((B, 32), jnp.float32),
            pltpu.VMEM((B, 16), jnp.float32),
            pltpu.VMEM((B, 32), jnp.float32),
            pltpu.VMEM((B, 32), jnp.float32),
            pltpu.VMEM((B, 32), jnp.float32),
            pltpu.VMEM((B, 32), jnp.float32),
            pltpu.VMEM_SHARED((NPAD, 32), jnp.float32),
        ] + [pltpu.SemaphoreType.DMA] * 12,
        compiler_params=pltpu.CompilerParams(
            needs_layout_passes=False, use_tc_tiling_on_sc=False),
    )


def _prep_body(x_ref, w1_ref, b1_ref, xnh_ref, xnb_ref):
    x = x_ref[...]
    h = jnp.maximum(x @ w1_ref[...] + b1_ref[...], 0.0)
    nrm = jnp.sqrt(jnp.sum(h * h, axis=1, keepdims=True))
    xn = h / jnp.maximum(nrm, 1e-12)
    pad2 = jnp.zeros((NPAD - N, 32), jnp.float32)
    pad1 = jnp.zeros((NPAD - N, 16), jnp.float32)
    xnh_ref[...] = jnp.concatenate([jnp.concatenate([xn, h], axis=1), pad2], axis=0)
    xnb_ref[...] = jnp.concatenate([xn, pad1], axis=0)


def _comb_body(nd_ref, beta_ref, x1_ref, xnh_ref, xnb_ref):
    nd = nd_ref[...]
    num = nd[0, :N, 0:16] + nd[1, :N, 0:16]
    den = nd[0, :N, 16:17] + nd[1, :N, 16:17]
    x1 = num / den
    x1_ref[...] = x1
    nrm = jnp.sqrt(jnp.sum(x1 * x1, axis=1, keepdims=True))
    xn = x1 / jnp.maximum(nrm, 1e-12)
    pad2 = jnp.zeros((NPAD - N, 32), jnp.float32)
    pad1 = jnp.zeros((NPAD - N, 16), jnp.float32)
    xnh_ref[...] = jnp.concatenate([jnp.concatenate([xn, x1], axis=1), pad2], axis=0)
    xnb_ref[...] = jnp.concatenate([xn * beta_ref[0, 0], pad1], axis=0)


def _final_body(nd_ref, w2_ref, b2_ref, out_ref):
    nd = nd_ref[...]
    num = nd[0, :N, 0:16] + nd[1, :N, 0:16]
    den = nd[0, :N, 16:17] + nd[1, :N, 16:17]
    h2 = num / den
    logits = h2 @ w2_ref[...] + b2_ref[...]
    m = jnp.max(logits, axis=1, keepdims=True)
    lse = jnp.log(jnp.sum(jnp.exp(logits - m), axis=1, keepdims=True)) + m
    out_ref[...] = logits - lse


_prep = pl.pallas_call(
    _prep_body,
    out_shape=[
        jax.ShapeDtypeStruct((NPAD, 32), jnp.float32),
        jax.ShapeDtypeStruct((NPAD, 16), jnp.float32),
    ],
)

_comb = pl.pallas_call(
    _comb_body,
    out_shape=[
        jax.ShapeDtypeStruct((N, 16), jnp.float32),
        jax.ShapeDtypeStruct((NPAD, 32), jnp.float32),
        jax.ShapeDtypeStruct((NPAD, 16), jnp.float32),
    ],
)

_final = pl.pallas_call(
    _final_body,
    out_shape=jax.ShapeDtypeStruct((N, C), jnp.float32),
)


def kernel(x, edge_index, W1, b1, beta2, W2, b2):
    loop = jnp.arange(N, dtype=jnp.int32)
    padi = N + (jnp.arange(ETP - E - N, dtype=jnp.int32) % (NPAD - N))
    src = jnp.concatenate([edge_index[0], loop, padi]).reshape(NW, NB, B)
    dst = jnp.concatenate([edge_index[1], loop, padi]).reshape(NW, NB, B)

    xnh1, xnb1 = _prep(x, W1, b1.reshape(1, H))
    ep = _edge_pass()
    nd1 = ep(src, dst, xnh1, xnb1)
    x1, xnh2, xnb2 = _comb(nd1, beta2.reshape(1, 1))
    nd2 = ep(src, dst, xnh2, xnb2)
    logp = _final(nd2, W2, b2.reshape(1, C))
    return (logp, x1)
